# baseline (speedup 1.0000x reference)
import functools

import jax
import jax.numpy as jnp
from jax import lax
from jax.experimental import pallas as pl
from jax.experimental.pallas import tpu as pltpu
from jax.experimental.pallas import tpu_sc as plsc

LVLS = 6
NBATCH = 16
NQ = 900
ND = 256
NCLS = 91
NFLAT = NQ * NCLS
NPAD = 81904
NVEC = NPAD // 16
KTOP = 100
OUTW = 112
HSTRIDE = 2048
NBINS1 = 2048
NBINS2 = 2048
NBINS3 = 1024


def _heads_body(x_ref, r_ref, wc_ref, bc_ref, w1_ref, b1_ref, w2_ref, b2_ref,
                w3_ref, b3_ref, cls_ref, coord_ref):
    h = x_ref[0]
    logits = jnp.dot(h, wc_ref[...]) + bc_ref[...]
    cls_ref[0] = logits[:, :NCLS]
    h1 = jnp.maximum(jnp.dot(h, w1_ref[...]) + b1_ref[...], 0.0)
    h2 = jnp.maximum(jnp.dot(h1, w2_ref[...]) + b2_ref[...], 0.0)
    t = jnp.dot(h2, w3_ref[...]) + b3_ref[...]
    r = jnp.clip(r_ref[0], 0.0, 1.0)
    inv = jnp.log(jnp.clip(r, 1e-5, None) / jnp.clip(1.0 - r, 1e-5, None))
    coord_ref[0] = jax.nn.sigmoid(t[:, :4] + inv)


def _run_heads(xr, rr, wcp, bcp, w1, b1r, w2, b2r, w3p, b3p):
    n = LVLS * NBATCH
    full = lambda s: pl.BlockSpec(s, lambda i: (0,) * len(s))
    return pl.pallas_call(
        _heads_body,
        grid=(n,),
        in_specs=[
            pl.BlockSpec((1, NQ, ND), lambda i: (i, 0, 0)),
            pl.BlockSpec((1, NQ, 4), lambda i: (i, 0, 0)),
            full((ND, 128)), full((1, 128)),
            full((ND, ND)), full((1, ND)),
            full((ND, ND)), full((1, ND)),
            full((ND, 128)), full((1, 128)),
        ],
        out_specs=[
            pl.BlockSpec((1, NQ, NCLS), lambda i: (i, 0, 0)),
            pl.BlockSpec((1, NQ, 4), lambda i: (i, 0, 0)),
        ],
        out_shape=[
            jax.ShapeDtypeStruct((n, NQ, NCLS), jnp.float32),
            jax.ShapeDtypeStruct((n, NQ, 4), jnp.float32),
        ],
        compiler_params=pltpu.CompilerParams(
            dimension_semantics=("arbitrary",)),
    )(xr, rr, wcp, bcp, w1, b1r, w2, b2r, w3p, b3p)


def _topk_body(bits_hbm, boxes_hbm, wv_hbm, hv_hbm, zz_hbm,
               scores_hbm, labels_hbm, boxout_hbm,
               keys_v, hist_v, tot_v, ck_v, ci_v, eqi_v,
               outs_u, outs_f, outl_v, rowb_v, box_v, wvec_v, hvec_v,
               outb_v):
    c = lax.axis_index("c")
    s = lax.axis_index("s")
    img = s * 2 + c

    @pl.when(img < NBATCH)
    def _work():
        iota = lax.iota(jnp.int32, 16)
        ones_i = jnp.ones((16,), jnp.int32)
        lane_base = iota * HSTRIDE

        pltpu.sync_copy(bits_hbm.at[img], keys_v)
        pltpu.sync_copy(boxes_hbm.at[img], box_v)
        pltpu.sync_copy(wv_hbm.at[img], wvec_v)
        pltpu.sync_copy(hv_hbm.at[img], hvec_v)

        def bcast_u32(x):
            return jnp.broadcast_to(x.astype(jnp.uint32), (16,))

        def bcast_i32(x):
            return jnp.broadcast_to(x.astype(jnp.int32), (16,))

        pltpu.sync_copy(zz_hbm, hist_v)

        def p1_body(i, carry):
            raw = keys_v[pl.ds(i * 16, 16)]
            sm = raw >> jnp.uint32(31)
            mm = (jnp.uint32(0) - sm) | jnp.uint32(0x80000000)
            key = raw ^ mm
            keys_v[pl.ds(i * 16, 16)] = key
            b1 = (key >> jnp.uint32(21)).astype(jnp.int32)
            plsc.addupdate_scatter(hist_v, [lane_base + b1], ones_i)
            return carry
        lax.fori_loop(0, NVEC, p1_body, 0)

        def _reduce_hist(nbins):
            def red_body(w, carry):
                acc = hist_v[pl.ds(w * 16, 16)]
                for l in range(1, 16):
                    acc = acc + hist_v[pl.ds(l * HSTRIDE + w * 16, 16)]
                tot_v[pl.ds(w * 16, 16)] = acc
                return carry
            lax.fori_loop(0, nbins // 16, red_body, 0)

        def _find_bin(nbins, kneed):
            def body(j, carry):
                above, fbin, cgt = carry
                v = nbins // 16 - 1 - j
                h = tot_v[pl.ds(v * 16, 16)]
                suff = lax.rev(jnp.cumsum(lax.rev(h, (0,)), axis=0), (0,))
                tota = above + suff
                m = (tota >= kneed)
                cnt = jnp.sum(m.astype(jnp.int32))
                lane = cnt - 1
                sel = (iota == lane)
                tot_l = jnp.sum(jnp.where(sel, tota, 0))
                h_l = jnp.sum(jnp.where(sel, h, 0))
                hit = jnp.logical_and(cnt > 0, fbin < 0)
                fbin = jnp.where(hit, v * 16 + lane, fbin)
                cgt = jnp.where(hit, tot_l - h_l, cgt)
                above = above + jnp.sum(h)
                return (above, fbin, cgt)
            _, fbin, cgt = lax.fori_loop(
                0, nbins // 16, body, (jnp.int32(0), jnp.int32(-1),
                                       jnp.int32(0)))
            return fbin, cgt

        _reduce_hist(NBINS1)
        b1f, cgt1 = _find_bin(NBINS1, jnp.int32(KTOP))
        kneed2 = jnp.int32(KTOP) - cgt1

        pltpu.sync_copy(zz_hbm, hist_v)
        p1vec = bcast_u32(b1f)

        def p2_body(i, carry):
            key = keys_v[pl.ds(i * 16, 16)]
            m = (key >> jnp.uint32(21)) == p1vec
            b2 = ((key >> jnp.uint32(10)) & jnp.uint32(0x7FF)).astype(jnp.int32)
            plsc.addupdate_scatter(hist_v, [lane_base + b2], ones_i, mask=m)
            return carry
        lax.fori_loop(0, NVEC, p2_body, 0)
        _reduce_hist(NBINS2)
        b2f, cgt2 = _find_bin(NBINS2, kneed2)
        kneed3 = kneed2 - cgt2

        pltpu.sync_copy(zz_hbm, hist_v)
        pref22 = (b1f << 11) | b2f
        p22vec = bcast_u32(pref22)

        def p3_body(i, carry):
            key = keys_v[pl.ds(i * 16, 16)]
            m = (key >> jnp.uint32(10)) == p22vec
            b3 = (key & jnp.uint32(0x3FF)).astype(jnp.int32)
            plsc.addupdate_scatter(hist_v, [lane_base + b3], ones_i, mask=m)
            return carry
        lax.fori_loop(0, NVEC, p3_body, 0)
        _reduce_hist(NBINS3)
        b3f, cgt3 = _find_bin(NBINS3, kneed3)

        tthr = ((b1f.astype(jnp.uint32) << jnp.uint32(21))
                | (b2f.astype(jnp.uint32) << jnp.uint32(10))
                | b3f.astype(jnp.uint32))
        cnt_gt = cgt1 + cgt2 + cgt3
        needed_eq = jnp.int32(KTOP) - cnt_gt
        tvec = jnp.broadcast_to(tthr, (16,))

        zu = jnp.zeros((16,), jnp.uint32)
        zi = jnp.zeros((16,), jnp.int32)
        for v in range(16):
            ck_v[pl.ds(v * 16, 16)] = zu
            ci_v[pl.ds(v * 16, 16)] = zi
        for v in range(OUTW // 16):
            outs_u[pl.ds(v * 16, 16)] = zi
            outl_v[pl.ds(v * 16, 16)] = zi
            rowb_v[pl.ds(v * 16, 16)] = zi

        def col_body(i, carry):
            og, oe = carry
            key = keys_v[pl.ds(i * 16, 16)]
            ge = key >= tvec
            ng = jnp.sum(ge.astype(jnp.int32))

            def slow(og, oe):
                idxv = i * 16 + iota
                gt = key > tvec
                cg = jnp.sum(gt.astype(jnp.int32))
                plsc.store_compressed(ck_v.at[pl.ds(og, 16)], key, mask=gt)
                plsc.store_compressed(ci_v.at[pl.ds(og, 16)], idxv, mask=gt)
                eq = key == tvec
                rank = jnp.cumsum(eq.astype(jnp.int32))
                keep = jnp.logical_and(eq, rank <= (needed_eq - oe))
                ce = jnp.sum(keep.astype(jnp.int32))
                plsc.store_compressed(eqi_v.at[pl.ds(oe, 16)], idxv, mask=keep)
                return (og + cg, oe + ce)

            return lax.cond(ng > 0, slow, lambda og, oe: (og, oe), og, oe)

        og, oe = lax.fori_loop(0, NVEC, col_body,
                               (jnp.int32(0), jnp.int32(0)))

        for j in range(7):
            m = (j * 16 + iota) < oe
            oldk = ck_v[pl.ds(og + j * 16, 16)]
            ck_v[pl.ds(og + j * 16, 16)] = jnp.where(m, tvec, oldk)
            ev = eqi_v[pl.ds(j * 16, 16)]
            oldi = ci_v[pl.ds(og + j * 16, 16)]
            ci_v[pl.ds(og + j * 16, 16)] = jnp.where(m, ev, oldi)

        lane0 = iota == 0
        big = jnp.broadcast_to(jnp.int32(0x7FFFFFFF), (16,))
        cand = tuple(ck_v[pl.ds(v * 16, 16)] for v in range(7)) + \
               tuple(ci_v[pl.ds(v * 16, 16)] for v in range(7))

        def sel_body(t, carry):
            ks = carry[:7]
            js = carry[7:]
            vm = ks[0]
            for j in range(1, 7):
                vm = jnp.maximum(vm, ks[j])
            g = jnp.max(vm)
            gvec = jnp.broadcast_to(g, (16,))
            im = jnp.where(ks[0] == gvec, js[0], big)
            for j in range(1, 7):
                im = jnp.minimum(im, jnp.where(ks[j] == gvec, js[j], big))
            gi = jnp.min(im)
            givec = bcast_i32(gi)
            tb = bcast_i32(t)
            plsc.store_scatter(outs_u, [tb], plsc.bitcast(gvec, jnp.int32),
                               mask=lane0)
            plsc.store_scatter(outl_v, [tb], givec % 91, mask=lane0)
            plsc.store_scatter(rowb_v, [tb], givec // 91, mask=lane0)
            newks = tuple(
                jnp.where(jnp.logical_and(ks[j] == gvec, js[j] == givec),
                          jnp.uint32(0), ks[j])
                for j in range(7))
            return newks + js

        lax.fori_loop(0, KTOP, sel_body, cand)

        for v in range(OUTW // 16):
            k = plsc.bitcast(outs_u[pl.ds(v * 16, 16)], jnp.uint32)
            posm = (k >> jnp.uint32(31)) == jnp.uint32(1)
            bits = jnp.where(posm, k & jnp.uint32(0x7FFFFFFF), ~k)
            x = plsc.bitcast(bits, jnp.float32)
            e = jnp.exp(-jnp.abs(x))
            outs_f[pl.ds(v * 16, 16)] = jnp.where(
                x >= 0, 1.0 / (1.0 + e), e / (1.0 + e))

        wv = wvec_v[...]
        hv = hvec_v[...]
        for v in range(OUTW // 16):
            r4 = rowb_v[pl.ds(v * 16, 16)] * 4
            cx = plsc.load_gather(box_v, [r4])
            cy = plsc.load_gather(box_v, [r4 + 1])
            w = plsc.load_gather(box_v, [r4 + 2])
            h = plsc.load_gather(box_v, [r4 + 3])
            pos4 = (v * 16 + iota) * 4
            plsc.store_scatter(outb_v, [pos4], (cx - 0.5 * w) * wv)
            plsc.store_scatter(outb_v, [pos4 + 1], (cy - 0.5 * h) * hv)
            plsc.store_scatter(outb_v, [pos4 + 2], (cx + 0.5 * w) * wv)
            plsc.store_scatter(outb_v, [pos4 + 3], (cy + 0.5 * h) * hv)

        pltpu.sync_copy(outs_f, scores_hbm.at[img])
        pltpu.sync_copy(outl_v, labels_hbm.at[img])
        pltpu.sync_copy(outb_v, boxout_hbm.at[img])


def _run_topk(bits, boxes5, wb, hb, zz):
    mesh = plsc.VectorSubcoreMesh(core_axis_name="c", subcore_axis_name="s",
                                  num_cores=2, num_subcores=16)
    fn = pl.kernel(
        _topk_body,
        out_type=[
            jax.ShapeDtypeStruct((NBATCH, OUTW), jnp.float32),
            jax.ShapeDtypeStruct((NBATCH, OUTW), jnp.int32),
            jax.ShapeDtypeStruct((NBATCH, OUTW * 4), jnp.float32),
        ],
        mesh=mesh,
        compiler_params=pltpu.CompilerParams(needs_layout_passes=False),
        scratch_types=[
            pltpu.VMEM((NPAD,), jnp.uint32),
            pltpu.VMEM((16 * HSTRIDE,), jnp.int32),
            pltpu.VMEM((HSTRIDE,), jnp.int32),
            pltpu.VMEM((256,), jnp.uint32),
            pltpu.VMEM((256,), jnp.int32),
            pltpu.VMEM((128,), jnp.int32),
            pltpu.VMEM((OUTW,), jnp.int32),
            pltpu.VMEM((OUTW,), jnp.float32),
            pltpu.VMEM((OUTW,), jnp.int32),
            pltpu.VMEM((OUTW,), jnp.int32),
            pltpu.VMEM((NQ * 4,), jnp.float32),
            pltpu.VMEM((16,), jnp.float32),
            pltpu.VMEM((16,), jnp.float32),
            pltpu.VMEM((OUTW * 4,), jnp.float32),
        ],
    )
    return fn(bits, boxes5, wb, hb, zz)


def kernel(x, init_reference, inter_references, orig_target_sizes,
           Wc, bc, W1, b1, W2, b2, W3, b3):
    n = LVLS * NBATCH
    xr = x.reshape(n, NQ, ND)
    rr = jnp.concatenate([init_reference[None], inter_references[:LVLS - 1]],
                         axis=0).reshape(n, NQ, 4)
    wcp = jnp.pad(Wc, ((0, 0), (0, 128 - NCLS)))
    bcp = jnp.pad(bc, (0, 128 - NCLS)).reshape(1, 128)
    w3p = jnp.pad(W3, ((0, 0), (0, 124)))
    b3p = jnp.pad(b3, (0, 124)).reshape(1, 128)
    cls, coord = _run_heads(xr, rr, wcp, bcp, W1, b1.reshape(1, ND),
                            W2, b2.reshape(1, ND), w3p, b3p)
    outputs_class = cls.reshape(LVLS, NBATCH, NQ, NCLS)
    outputs_coord = coord.reshape(LVLS, NBATCH, NQ, 4)

    logits5 = cls.reshape(LVLS, NBATCH, NFLAT)[LVLS - 1]
    pad = jnp.full((NBATCH, NPAD - NFLAT), -jnp.inf, jnp.float32)
    bits = lax.bitcast_convert_type(
        jnp.concatenate([logits5, pad], axis=1), jnp.uint32)
    boxes5 = outputs_coord[LVLS - 1].reshape(NBATCH, NQ * 4)
    img_h = orig_target_sizes[:, 0].astype(jnp.float32)
    img_w = orig_target_sizes[:, 1].astype(jnp.float32)
    wb = jnp.broadcast_to(img_w[:, None], (NBATCH, 16)) * jnp.ones((NBATCH, 16))
    hb = jnp.broadcast_to(img_h[:, None], (NBATCH, 16)) * jnp.ones((NBATCH, 16))
    zz = jnp.zeros((16 * HSTRIDE,), jnp.int32)
    scores_p, labels_p, boxes_p = _run_topk(bits, boxes5, wb, hb, zz)
    boxes_p = boxes_p.reshape(NBATCH, OUTW, 4)
    return (outputs_class, outputs_coord,
            scores_p[:, :KTOP], labels_p[:, :KTOP], boxes_p[:, :KTOP, :])


# speedup vs baseline: 1.2444x; 1.2444x over previous
import functools

import jax
import jax.numpy as jnp
from jax import lax
from jax.experimental import pallas as pl
from jax.experimental.pallas import tpu as pltpu
from jax.experimental.pallas import tpu_sc as plsc

LVLS = 6
NBATCH = 16
NQ = 900
ND = 256
NCLS = 91
NFLAT = NQ * NCLS
NPAD = 81920
NVEC = NPAD // 16
KTOP = 100
OUTW = 112
HSTRIDE = 2048
NBINS1 = 2048
NBINS2 = 2048
NBINS3 = 1024


def _heads_body(x_ref, r_ref, wc_ref, bc_ref, w1_ref, b1_ref, w2_ref, b2_ref,
                w3_ref, b3_ref, cls_ref, coord_ref):
    h = x_ref[0]
    logits = jnp.dot(h, wc_ref[...]) + bc_ref[...]
    cls_ref[0] = logits[:, :NCLS]
    h1 = jnp.maximum(jnp.dot(h, w1_ref[...]) + b1_ref[...], 0.0)
    h2 = jnp.maximum(jnp.dot(h1, w2_ref[...]) + b2_ref[...], 0.0)
    t = jnp.dot(h2, w3_ref[...]) + b3_ref[...]
    r = jnp.clip(r_ref[0], 0.0, 1.0)
    inv = jnp.log(jnp.clip(r, 1e-5, None) / jnp.clip(1.0 - r, 1e-5, None))
    coord_ref[0] = jax.nn.sigmoid(t[:, :4] + inv)


def _run_heads(xr, rr, wcp, bcp, w1, b1r, w2, b2r, w3p, b3p):
    n = xr.shape[0]
    full = lambda s: pl.BlockSpec(s, lambda i: (0,) * len(s))
    return pl.pallas_call(
        _heads_body,
        grid=(n,),
        in_specs=[
            pl.BlockSpec((1, NQ, ND), lambda i: (i, 0, 0)),
            pl.BlockSpec((1, NQ, 4), lambda i: (i, 0, 0)),
            full((ND, 128)), full((1, 128)),
            full((ND, ND)), full((1, ND)),
            full((ND, ND)), full((1, ND)),
            full((ND, 128)), full((1, 128)),
        ],
        out_specs=[
            pl.BlockSpec((1, NQ, NCLS), lambda i: (i, 0, 0)),
            pl.BlockSpec((1, NQ, 4), lambda i: (i, 0, 0)),
        ],
        out_shape=[
            jax.ShapeDtypeStruct((n, NQ, NCLS), jnp.float32),
            jax.ShapeDtypeStruct((n, NQ, 4), jnp.float32),
        ],
        compiler_params=pltpu.CompilerParams(
            dimension_semantics=("arbitrary",)),
    )(xr, rr, wcp, bcp, w1, b1r, w2, b2r, w3p, b3p)


def _topk_body(bits_hbm, boxes_hbm, wv_hbm, hv_hbm, zz_hbm,
               scores_hbm, labels_hbm, boxout_hbm,
               keys_v, hist_v, tot_v, ck_v, ci_v, eqi_v,
               outs_u, outs_f, outl_v, rowb_v, box_v, wvec_v, hvec_v,
               outb_v):
    c = lax.axis_index("c")
    s = lax.axis_index("s")
    img = s * 2 + c

    @pl.when(img < NBATCH)
    def _work():
        iota = lax.iota(jnp.int32, 16)
        ones_i = jnp.ones((16,), jnp.int32)
        lane_base = iota * HSTRIDE

        pltpu.sync_copy(bits_hbm.at[img], keys_v)
        pltpu.sync_copy(boxes_hbm.at[img], box_v)
        pltpu.sync_copy(wv_hbm.at[img], wvec_v)
        pltpu.sync_copy(hv_hbm.at[img], hvec_v)

        def bcast_u32(x):
            return jnp.broadcast_to(x.astype(jnp.uint32), (16,))

        def bcast_i32(x):
            return jnp.broadcast_to(x.astype(jnp.int32), (16,))

        pltpu.sync_copy(zz_hbm, hist_v)

        @plsc.parallel_loop(0, NVEC, 1, unroll=8)
        def p1_body(i):
            raw = plsc.bitcast(keys_v[pl.ds(i * 16, 16)], jnp.uint32)
            sm = raw >> jnp.uint32(31)
            mm = (jnp.uint32(0) - sm) | jnp.uint32(0x80000000)
            key = raw ^ mm
            keys_v[pl.ds(i * 16, 16)] = plsc.bitcast(key, jnp.float32)
            b1 = (key >> jnp.uint32(21)).astype(jnp.int32)
            plsc.addupdate_scatter(hist_v, [lane_base + b1], ones_i)

        def _reduce_hist(nbins):
            def red_body(w, carry):
                acc = hist_v[pl.ds(w * 16, 16)]
                for l in range(1, 16):
                    acc = acc + hist_v[pl.ds(l * HSTRIDE + w * 16, 16)]
                tot_v[pl.ds(w * 16, 16)] = acc
                return carry
            lax.fori_loop(0, nbins // 16, red_body, 0)

        def _find_bin(nbins, kneed):
            def body(j, carry):
                above, fbin, cgt = carry
                v = nbins // 16 - 1 - j
                h = tot_v[pl.ds(v * 16, 16)]
                suff = lax.rev(jnp.cumsum(lax.rev(h, (0,)), axis=0), (0,))
                tota = above + suff
                m = (tota >= kneed)
                cnt = jnp.sum(m.astype(jnp.int32))
                lane = cnt - 1
                sel = (iota == lane)
                tot_l = jnp.sum(jnp.where(sel, tota, 0))
                h_l = jnp.sum(jnp.where(sel, h, 0))
                hit = jnp.logical_and(cnt > 0, fbin < 0)
                fbin = jnp.where(hit, v * 16 + lane, fbin)
                cgt = jnp.where(hit, tot_l - h_l, cgt)
                above = above + jnp.sum(h)
                return (above, fbin, cgt)
            _, fbin, cgt = lax.fori_loop(
                0, nbins // 16, body, (jnp.int32(0), jnp.int32(-1),
                                       jnp.int32(0)))
            return fbin, cgt

        _reduce_hist(NBINS1)
        b1f, cgt1 = _find_bin(NBINS1, jnp.int32(KTOP))
        kneed2 = jnp.int32(KTOP) - cgt1

        pltpu.sync_copy(zz_hbm, hist_v)
        p1vec = bcast_u32(b1f)

        @plsc.parallel_loop(0, NVEC, 1, unroll=8)
        def p2_body(i):
            key = plsc.bitcast(keys_v[pl.ds(i * 16, 16)], jnp.uint32)
            m = (key >> jnp.uint32(21)) == p1vec
            b2 = ((key >> jnp.uint32(10)) & jnp.uint32(0x7FF)).astype(jnp.int32)
            plsc.addupdate_scatter(hist_v, [lane_base + b2], ones_i, mask=m)
        _reduce_hist(NBINS2)
        b2f, cgt2 = _find_bin(NBINS2, kneed2)
        kneed3 = kneed2 - cgt2

        pltpu.sync_copy(zz_hbm, hist_v)
        pref22 = (b1f << 11) | b2f
        p22vec = bcast_u32(pref22)

        @plsc.parallel_loop(0, NVEC, 1, unroll=8)
        def p3_body(i):
            key = plsc.bitcast(keys_v[pl.ds(i * 16, 16)], jnp.uint32)
            m = (key >> jnp.uint32(10)) == p22vec
            b3 = (key & jnp.uint32(0x3FF)).astype(jnp.int32)
            plsc.addupdate_scatter(hist_v, [lane_base + b3], ones_i, mask=m)
        _reduce_hist(NBINS3)
        b3f, cgt3 = _find_bin(NBINS3, kneed3)

        tthr = ((b1f.astype(jnp.uint32) << jnp.uint32(21))
                | (b2f.astype(jnp.uint32) << jnp.uint32(10))
                | b3f.astype(jnp.uint32))
        cnt_gt = cgt1 + cgt2 + cgt3
        needed_eq = jnp.int32(KTOP) - cnt_gt
        tvec = jnp.broadcast_to(tthr, (16,))

        zu = jnp.zeros((16,), jnp.uint32)
        zi = jnp.zeros((16,), jnp.int32)
        for v in range(16):
            ck_v[pl.ds(v * 16, 16)] = zu
            ci_v[pl.ds(v * 16, 16)] = zi
        for v in range(OUTW // 16):
            outs_u[pl.ds(v * 16, 16)] = zi
            outl_v[pl.ds(v * 16, 16)] = zi
            rowb_v[pl.ds(v * 16, 16)] = zi

        def col_body(i, carry):
            og, oe = carry
            key = plsc.bitcast(keys_v[pl.ds(i * 16, 16)], jnp.uint32)
            ge = key >= tvec
            ng = jnp.sum(ge.astype(jnp.int32))

            def slow(og, oe):
                idxv = i * 16 + iota
                gt = key > tvec
                cg = jnp.sum(gt.astype(jnp.int32))
                plsc.store_compressed(ck_v.at[pl.ds(og, 16)], key, mask=gt)
                plsc.store_compressed(ci_v.at[pl.ds(og, 16)], idxv, mask=gt)
                eq = key == tvec
                rank = jnp.cumsum(eq.astype(jnp.int32))
                keep = jnp.logical_and(eq, rank <= (needed_eq - oe))
                ce = jnp.sum(keep.astype(jnp.int32))
                plsc.store_compressed(eqi_v.at[pl.ds(oe, 16)], idxv, mask=keep)
                return (og + cg, oe + ce)

            return lax.cond(ng > 0, slow, lambda og, oe: (og, oe), og, oe)

        og, oe = lax.fori_loop(0, NVEC, col_body,
                               (jnp.int32(0), jnp.int32(0)))

        for j in range(7):
            m = (j * 16 + iota) < oe
            oldk = ck_v[pl.ds(og + j * 16, 16)]
            ck_v[pl.ds(og + j * 16, 16)] = jnp.where(m, tvec, oldk)
            ev = eqi_v[pl.ds(j * 16, 16)]
            oldi = ci_v[pl.ds(og + j * 16, 16)]
            ci_v[pl.ds(og + j * 16, 16)] = jnp.where(m, ev, oldi)

        lane0 = iota == 0
        big = jnp.broadcast_to(jnp.int32(0x7FFFFFFF), (16,))
        cand = tuple(ck_v[pl.ds(v * 16, 16)] for v in range(7)) + \
               tuple(ci_v[pl.ds(v * 16, 16)] for v in range(7))

        def sel_body(t, carry):
            ks = carry[:7]
            js = carry[7:]
            vm = ks[0]
            for j in range(1, 7):
                vm = jnp.maximum(vm, ks[j])
            g = jnp.max(vm)
            gvec = jnp.broadcast_to(g, (16,))
            im = jnp.where(ks[0] == gvec, js[0], big)
            for j in range(1, 7):
                im = jnp.minimum(im, jnp.where(ks[j] == gvec, js[j], big))
            gi = jnp.min(im)
            givec = bcast_i32(gi)
            tb = bcast_i32(t)
            plsc.store_scatter(outs_u, [tb], plsc.bitcast(gvec, jnp.int32),
                               mask=lane0)
            plsc.store_scatter(outl_v, [tb], givec % 91, mask=lane0)
            plsc.store_scatter(rowb_v, [tb], givec // 91, mask=lane0)
            newks = tuple(
                jnp.where(jnp.logical_and(ks[j] == gvec, js[j] == givec),
                          jnp.uint32(0), ks[j])
                for j in range(7))
            return newks + js

        lax.fori_loop(0, KTOP, sel_body, cand)

        for v in range(OUTW // 16):
            k = plsc.bitcast(outs_u[pl.ds(v * 16, 16)], jnp.uint32)
            posm = (k >> jnp.uint32(31)) == jnp.uint32(1)
            bits = jnp.where(posm, k & jnp.uint32(0x7FFFFFFF), ~k)
            x = plsc.bitcast(bits, jnp.float32)
            e = jnp.exp(-jnp.abs(x))
            outs_f[pl.ds(v * 16, 16)] = jnp.where(
                x >= 0, 1.0 / (1.0 + e), e / (1.0 + e))

        wv = wvec_v[...]
        hv = hvec_v[...]
        for v in range(OUTW // 16):
            r4 = rowb_v[pl.ds(v * 16, 16)] * 4
            cx = plsc.load_gather(box_v, [r4])
            cy = plsc.load_gather(box_v, [r4 + 1])
            w = plsc.load_gather(box_v, [r4 + 2])
            h = plsc.load_gather(box_v, [r4 + 3])
            pos4 = (v * 16 + iota) * 4
            plsc.store_scatter(outb_v, [pos4], (cx - 0.5 * w) * wv)
            plsc.store_scatter(outb_v, [pos4 + 1], (cy - 0.5 * h) * hv)
            plsc.store_scatter(outb_v, [pos4 + 2], (cx + 0.5 * w) * wv)
            plsc.store_scatter(outb_v, [pos4 + 3], (cy + 0.5 * h) * hv)

        pltpu.sync_copy(outs_f, scores_hbm.at[img])
        pltpu.sync_copy(outl_v, labels_hbm.at[img])
        pltpu.sync_copy(outb_v, boxout_hbm.at[img])


def _run_topk(bits, boxes5, wb, hb, zz):
    mesh = plsc.VectorSubcoreMesh(core_axis_name="c", subcore_axis_name="s",
                                  num_cores=2, num_subcores=16)
    fn = pl.kernel(
        _topk_body,
        out_type=[
            jax.ShapeDtypeStruct((NBATCH, OUTW), jnp.float32),
            jax.ShapeDtypeStruct((NBATCH, OUTW), jnp.int32),
            jax.ShapeDtypeStruct((NBATCH, OUTW * 4), jnp.float32),
        ],
        mesh=mesh,
        compiler_params=pltpu.CompilerParams(needs_layout_passes=False),
        scratch_types=[
            pltpu.VMEM((NPAD,), jnp.float32),
            pltpu.VMEM((16 * HSTRIDE,), jnp.int32),
            pltpu.VMEM((HSTRIDE,), jnp.int32),
            pltpu.VMEM((256,), jnp.uint32),
            pltpu.VMEM((256,), jnp.int32),
            pltpu.VMEM((128,), jnp.int32),
            pltpu.VMEM((OUTW,), jnp.int32),
            pltpu.VMEM((OUTW,), jnp.float32),
            pltpu.VMEM((OUTW,), jnp.int32),
            pltpu.VMEM((OUTW,), jnp.int32),
            pltpu.VMEM((NQ * 4,), jnp.float32),
            pltpu.VMEM((16,), jnp.float32),
            pltpu.VMEM((16,), jnp.float32),
            pltpu.VMEM((OUTW * 4,), jnp.float32),
        ],
    )
    return fn(bits, boxes5, wb, hb, zz)


def kernel(x, init_reference, inter_references, orig_target_sizes,
           Wc, bc, W1, b1, W2, b2, W3, b3):
    wcp = jnp.pad(Wc, ((0, 0), (0, 128 - NCLS)))
    bcp = jnp.pad(bc, (0, 128 - NCLS)).reshape(1, 128)
    w3p = jnp.pad(W3, ((0, 0), (0, 124)))
    b3p = jnp.pad(b3, (0, 124)).reshape(1, 128)
    b1r = b1.reshape(1, ND)
    b2r = b2.reshape(1, ND)

    cls5, coord5 = _run_heads(x[LVLS - 1], inter_references[LVLS - 2],
                              wcp, bcp, W1, b1r, W2, b2r, w3p, b3p)
    xr = x[:LVLS - 1].reshape((LVLS - 1) * NBATCH, NQ, ND)
    rr = jnp.concatenate([init_reference[None], inter_references[:LVLS - 2]],
                         axis=0).reshape((LVLS - 1) * NBATCH, NQ, 4)
    cls04, coord04 = _run_heads(xr, rr, wcp, bcp, W1, b1r, W2, b2r, w3p, b3p)
    outputs_class = jnp.concatenate(
        [cls04.reshape(LVLS - 1, NBATCH, NQ, NCLS), cls5[None]], axis=0)
    outputs_coord = jnp.concatenate(
        [coord04.reshape(LVLS - 1, NBATCH, NQ, 4), coord5[None]], axis=0)

    logits5 = cls5.reshape(NBATCH, NFLAT)
    pad = jnp.full((NBATCH, NPAD - NFLAT), -jnp.inf, jnp.float32)
    bits = jnp.concatenate([logits5, pad], axis=1)
    boxes5 = coord5.reshape(NBATCH, NQ * 4)
    img_h = orig_target_sizes[:, 0].astype(jnp.float32)
    img_w = orig_target_sizes[:, 1].astype(jnp.float32)
    wb = jnp.broadcast_to(img_w[:, None], (NBATCH, 16)) * jnp.ones((NBATCH, 16))
    hb = jnp.broadcast_to(img_h[:, None], (NBATCH, 16)) * jnp.ones((NBATCH, 16))
    zz = jnp.zeros((16 * HSTRIDE,), jnp.int32)
    scores_p, labels_p, boxes_p = _run_topk(bits, boxes5, wb, hb, zz)
    boxes_p = boxes_p.reshape(NBATCH, OUTW, 4)
    return (outputs_class, outputs_coord,
            scores_p[:, :KTOP], labels_p[:, :KTOP], boxes_p[:, :KTOP, :])


# speedup vs baseline: 1.3408x; 1.0774x over previous
import functools

import jax
import jax.numpy as jnp
from jax import lax
from jax.experimental import pallas as pl
from jax.experimental.pallas import tpu as pltpu
from jax.experimental.pallas import tpu_sc as plsc

LVLS = 6
NBATCH = 16
NQ = 900
ND = 256
NCLS = 91
NFLAT = NQ * NCLS
NPAD = 81920
NVEC = NPAD // 16
KTOP = 100
OUTW = 112
HSTRIDE = 2048
NBINS1 = 2048
NBINS2 = 2048
NBINS3 = 1024


def _heads_body(x_ref, r_ref, wc_ref, bc_ref, w1_ref, b1_ref, w2_ref, b2_ref,
                w3_ref, b3_ref, cls_ref, coord_ref):
    h = x_ref[0, 0]
    logits = jnp.dot(h, wc_ref[...]) + bc_ref[...]
    cls_ref[0] = logits[:, :NCLS]
    h1 = jnp.maximum(jnp.dot(h, w1_ref[...]) + b1_ref[...], 0.0)
    h2 = jnp.maximum(jnp.dot(h1, w2_ref[...]) + b2_ref[...], 0.0)
    t = jnp.dot(h2, w3_ref[...]) + b3_ref[...]
    r = jnp.clip(r_ref[0], 0.0, 1.0)
    inv = jnp.log(jnp.clip(r, 1e-5, None) / jnp.clip(1.0 - r, 1e-5, None))
    coord_ref[0] = jax.nn.sigmoid(t[:, :4] + inv)


def _run_heads(x4d, rr, wcp, bcp, w1, b1r, w2, b2r, w3p, b3p, lvl_lo, lvl_hi):
    nlvl = lvl_hi - lvl_lo
    n = nlvl * NBATCH
    full = lambda s: pl.BlockSpec(s, lambda i: (0,) * len(s))
    return pl.pallas_call(
        _heads_body,
        grid=(n,),
        in_specs=[
            pl.BlockSpec((1, 1, NQ, ND),
                         lambda i: (lvl_lo + i // NBATCH, i % NBATCH, 0, 0)),
            pl.BlockSpec((1, NQ, 4), lambda i: (lvl_lo * NBATCH + i, 0, 0)),
            full((ND, 128)), full((1, 128)),
            full((ND, ND)), full((1, ND)),
            full((ND, ND)), full((1, ND)),
            full((ND, 128)), full((1, 128)),
        ],
        out_specs=[
            pl.BlockSpec((1, NQ, NCLS), lambda i: (i, 0, 0)),
            pl.BlockSpec((1, NQ, 4), lambda i: (i, 0, 0)),
        ],
        out_shape=[
            jax.ShapeDtypeStruct((n, NQ, NCLS), jnp.float32),
            jax.ShapeDtypeStruct((n, NQ, 4), jnp.float32),
        ],
        compiler_params=pltpu.CompilerParams(
            dimension_semantics=("arbitrary",)),
    )(x4d, rr, wcp, bcp, w1, b1r, w2, b2r, w3p, b3p)


def _topk_body(bits_hbm, boxes_hbm, wv_hbm, hv_hbm, zz_hbm,
               scores_hbm, labels_hbm, boxout_hbm,
               keys_v, hist_v, tot_v, ck_v, ci_v, eqi_v,
               outs_u, outs_f, outl_v, rowb_v, box_v, wvec_v, hvec_v,
               outb_v):
    c = lax.axis_index("c")
    s = lax.axis_index("s")
    img = s * 2 + c

    @pl.when(img < NBATCH)
    def _work():
        iota = lax.iota(jnp.int32, 16)
        ones_i = jnp.ones((16,), jnp.int32)
        lane_base = iota * HSTRIDE

        pltpu.sync_copy(bits_hbm.at[img], keys_v)
        pltpu.sync_copy(boxes_hbm.at[img], box_v)
        pltpu.sync_copy(wv_hbm.at[img], wvec_v)
        pltpu.sync_copy(hv_hbm.at[img], hvec_v)

        def bcast_u32(x):
            return jnp.broadcast_to(x.astype(jnp.uint32), (16,))

        def bcast_i32(x):
            return jnp.broadcast_to(x.astype(jnp.int32), (16,))

        pltpu.sync_copy(zz_hbm, hist_v)

        @plsc.parallel_loop(0, NVEC, 1, unroll=8)
        def p1_body(i):
            raw = plsc.bitcast(keys_v[pl.ds(i * 16, 16)], jnp.uint32)
            sm = raw >> jnp.uint32(31)
            mm = (jnp.uint32(0) - sm) | jnp.uint32(0x80000000)
            key = raw ^ mm
            keys_v[pl.ds(i * 16, 16)] = plsc.bitcast(key, jnp.float32)
            b1 = (key >> jnp.uint32(21)).astype(jnp.int32)
            plsc.addupdate_scatter(hist_v, [lane_base + b1], ones_i)

        def _reduce_hist(nbins):
            def red_body(w, carry):
                acc = hist_v[pl.ds(w * 16, 16)]
                for l in range(1, 16):
                    acc = acc + hist_v[pl.ds(l * HSTRIDE + w * 16, 16)]
                tot_v[pl.ds(w * 16, 16)] = acc
                return carry
            lax.fori_loop(0, nbins // 16, red_body, 0)

        def _find_bin(nbins, kneed):
            def body(j, carry):
                above, fbin, cgt = carry
                v = nbins // 16 - 1 - j
                h = tot_v[pl.ds(v * 16, 16)]
                suff = lax.rev(jnp.cumsum(lax.rev(h, (0,)), axis=0), (0,))
                tota = above + suff
                m = (tota >= kneed)
                cnt = jnp.sum(m.astype(jnp.int32))
                lane = cnt - 1
                sel = (iota == lane)
                tot_l = jnp.sum(jnp.where(sel, tota, 0))
                h_l = jnp.sum(jnp.where(sel, h, 0))
                hit = jnp.logical_and(cnt > 0, fbin < 0)
                fbin = jnp.where(hit, v * 16 + lane, fbin)
                cgt = jnp.where(hit, tot_l - h_l, cgt)
                above = above + jnp.sum(h)
                return (above, fbin, cgt)
            _, fbin, cgt = lax.fori_loop(
                0, nbins // 16, body, (jnp.int32(0), jnp.int32(-1),
                                       jnp.int32(0)))
            return fbin, cgt

        _reduce_hist(NBINS1)
        b1f, cgt1 = _find_bin(NBINS1, jnp.int32(KTOP))
        kneed2 = jnp.int32(KTOP) - cgt1

        pltpu.sync_copy(zz_hbm, hist_v)
        p1vec = bcast_u32(b1f)

        @plsc.parallel_loop(0, NVEC, 1, unroll=8)
        def p2_body(i):
            key = plsc.bitcast(keys_v[pl.ds(i * 16, 16)], jnp.uint32)
            m = (key >> jnp.uint32(21)) == p1vec
            b2 = ((key >> jnp.uint32(10)) & jnp.uint32(0x7FF)).astype(jnp.int32)
            plsc.addupdate_scatter(hist_v, [lane_base + b2], ones_i, mask=m)
        _reduce_hist(NBINS2)
        b2f, cgt2 = _find_bin(NBINS2, kneed2)
        kneed3 = kneed2 - cgt2

        pltpu.sync_copy(zz_hbm, hist_v)
        pref22 = (b1f << 11) | b2f
        p22vec = bcast_u32(pref22)

        @plsc.parallel_loop(0, NVEC, 1, unroll=8)
        def p3_body(i):
            key = plsc.bitcast(keys_v[pl.ds(i * 16, 16)], jnp.uint32)
            m = (key >> jnp.uint32(10)) == p22vec
            b3 = (key & jnp.uint32(0x3FF)).astype(jnp.int32)
            plsc.addupdate_scatter(hist_v, [lane_base + b3], ones_i, mask=m)
        _reduce_hist(NBINS3)
        b3f, cgt3 = _find_bin(NBINS3, kneed3)

        tthr = ((b1f.astype(jnp.uint32) << jnp.uint32(21))
                | (b2f.astype(jnp.uint32) << jnp.uint32(10))
                | b3f.astype(jnp.uint32))
        cnt_gt = cgt1 + cgt2 + cgt3
        needed_eq = jnp.int32(KTOP) - cnt_gt
        tvec = jnp.broadcast_to(tthr, (16,))

        zu = jnp.zeros((16,), jnp.uint32)
        zi = jnp.zeros((16,), jnp.int32)
        for v in range(16):
            ck_v[pl.ds(v * 16, 16)] = zu
            ci_v[pl.ds(v * 16, 16)] = zi
        for v in range(OUTW // 16):
            outs_u[pl.ds(v * 16, 16)] = zi
            outl_v[pl.ds(v * 16, 16)] = zi
            rowb_v[pl.ds(v * 16, 16)] = zi

        def col_body(i, carry):
            og, oe = carry
            key = plsc.bitcast(keys_v[pl.ds(i * 16, 16)], jnp.uint32)
            ge = key >= tvec
            ng = jnp.sum(ge.astype(jnp.int32))

            def slow(og, oe):
                idxv = i * 16 + iota
                gt = key > tvec
                cg = jnp.sum(gt.astype(jnp.int32))
                plsc.store_compressed(ck_v.at[pl.ds(og, 16)], key, mask=gt)
                plsc.store_compressed(ci_v.at[pl.ds(og, 16)], idxv, mask=gt)
                eq = key == tvec
                rank = jnp.cumsum(eq.astype(jnp.int32))
                keep = jnp.logical_and(eq, rank <= (needed_eq - oe))
                ce = jnp.sum(keep.astype(jnp.int32))
                plsc.store_compressed(eqi_v.at[pl.ds(oe, 16)], idxv, mask=keep)
                return (og + cg, oe + ce)

            return lax.cond(ng > 0, slow, lambda og, oe: (og, oe), og, oe)

        og, oe = lax.fori_loop(0, NVEC, col_body,
                               (jnp.int32(0), jnp.int32(0)))

        for j in range(7):
            m = (j * 16 + iota) < oe
            oldk = ck_v[pl.ds(og + j * 16, 16)]
            ck_v[pl.ds(og + j * 16, 16)] = jnp.where(m, tvec, oldk)
            ev = eqi_v[pl.ds(j * 16, 16)]
            oldi = ci_v[pl.ds(og + j * 16, 16)]
            ci_v[pl.ds(og + j * 16, 16)] = jnp.where(m, ev, oldi)

        lane0 = iota == 0
        big = jnp.broadcast_to(jnp.int32(0x7FFFFFFF), (16,))
        cand = tuple(ck_v[pl.ds(v * 16, 16)] for v in range(7)) + \
               tuple(ci_v[pl.ds(v * 16, 16)] for v in range(7))

        def sel_body(t, carry):
            ks = carry[:7]
            js = carry[7:]
            vm = ks[0]
            for j in range(1, 7):
                vm = jnp.maximum(vm, ks[j])
            g = jnp.max(vm)
            gvec = jnp.broadcast_to(g, (16,))
            im = jnp.where(ks[0] == gvec, js[0], big)
            for j in range(1, 7):
                im = jnp.minimum(im, jnp.where(ks[j] == gvec, js[j], big))
            gi = jnp.min(im)
            givec = bcast_i32(gi)
            tb = bcast_i32(t)
            plsc.store_scatter(outs_u, [tb], plsc.bitcast(gvec, jnp.int32),
                               mask=lane0)
            plsc.store_scatter(outl_v, [tb], givec % 91, mask=lane0)
            plsc.store_scatter(rowb_v, [tb], givec // 91, mask=lane0)
            newks = tuple(
                jnp.where(jnp.logical_and(ks[j] == gvec, js[j] == givec),
                          jnp.uint32(0), ks[j])
                for j in range(7))
            return newks + js

        lax.fori_loop(0, KTOP, sel_body, cand)

        for v in range(OUTW // 16):
            k = plsc.bitcast(outs_u[pl.ds(v * 16, 16)], jnp.uint32)
            posm = (k >> jnp.uint32(31)) == jnp.uint32(1)
            bits = jnp.where(posm, k & jnp.uint32(0x7FFFFFFF), ~k)
            x = plsc.bitcast(bits, jnp.float32)
            e = jnp.exp(-jnp.abs(x))
            outs_f[pl.ds(v * 16, 16)] = jnp.where(
                x >= 0, 1.0 / (1.0 + e), e / (1.0 + e))

        wv = wvec_v[...]
        hv = hvec_v[...]
        for v in range(OUTW // 16):
            r4 = rowb_v[pl.ds(v * 16, 16)] * 4
            cx = plsc.load_gather(box_v, [r4])
            cy = plsc.load_gather(box_v, [r4 + 1])
            w = plsc.load_gather(box_v, [r4 + 2])
            h = plsc.load_gather(box_v, [r4 + 3])
            pos4 = (v * 16 + iota) * 4
            plsc.store_scatter(outb_v, [pos4], (cx - 0.5 * w) * wv)
            plsc.store_scatter(outb_v, [pos4 + 1], (cy - 0.5 * h) * hv)
            plsc.store_scatter(outb_v, [pos4 + 2], (cx + 0.5 * w) * wv)
            plsc.store_scatter(outb_v, [pos4 + 3], (cy + 0.5 * h) * hv)

        pltpu.sync_copy(outs_f, scores_hbm.at[img])
        pltpu.sync_copy(outl_v, labels_hbm.at[img])
        pltpu.sync_copy(outb_v, boxout_hbm.at[img])


def _run_topk(bits, boxes5, wb, hb, zz):
    mesh = plsc.VectorSubcoreMesh(core_axis_name="c", subcore_axis_name="s",
                                  num_cores=2, num_subcores=16)
    fn = pl.kernel(
        _topk_body,
        out_type=[
            jax.ShapeDtypeStruct((NBATCH, OUTW), jnp.float32),
            jax.ShapeDtypeStruct((NBATCH, OUTW), jnp.int32),
            jax.ShapeDtypeStruct((NBATCH, OUTW * 4), jnp.float32),
        ],
        mesh=mesh,
        compiler_params=pltpu.CompilerParams(needs_layout_passes=False),
        scratch_types=[
            pltpu.VMEM((NPAD,), jnp.float32),
            pltpu.VMEM((16 * HSTRIDE,), jnp.int32),
            pltpu.VMEM((HSTRIDE,), jnp.int32),
            pltpu.VMEM((256,), jnp.uint32),
            pltpu.VMEM((256,), jnp.int32),
            pltpu.VMEM((128,), jnp.int32),
            pltpu.VMEM((OUTW,), jnp.int32),
            pltpu.VMEM((OUTW,), jnp.float32),
            pltpu.VMEM((OUTW,), jnp.int32),
            pltpu.VMEM((OUTW,), jnp.int32),
            pltpu.VMEM((NQ * 4,), jnp.float32),
            pltpu.VMEM((16,), jnp.float32),
            pltpu.VMEM((16,), jnp.float32),
            pltpu.VMEM((OUTW * 4,), jnp.float32),
        ],
    )
    return fn(bits, boxes5, wb, hb, zz)


def kernel(x, init_reference, inter_references, orig_target_sizes,
           Wc, bc, W1, b1, W2, b2, W3, b3):
    wcp = jnp.pad(Wc, ((0, 0), (0, 128 - NCLS)))
    bcp = jnp.pad(bc, (0, 128 - NCLS)).reshape(1, 128)
    w3p = jnp.pad(W3, ((0, 0), (0, 124)))
    b3p = jnp.pad(b3, (0, 124)).reshape(1, 128)
    b1r = b1.reshape(1, ND)
    b2r = b2.reshape(1, ND)

    rr = jnp.concatenate([init_reference[None], inter_references[:LVLS - 1]],
                         axis=0).reshape(LVLS * NBATCH, NQ, 4)

    cls5, coord5 = _run_heads(x, rr, wcp, bcp, W1, b1r, W2, b2r, w3p, b3p,
                              LVLS - 1, LVLS)
    cls04, coord04 = _run_heads(x, rr, wcp, bcp, W1, b1r, W2, b2r, w3p, b3p,
                                0, LVLS - 1)
    outputs_class = jnp.concatenate(
        [cls04.reshape(LVLS - 1, NBATCH, NQ, NCLS), cls5[None]], axis=0)
    outputs_coord = jnp.concatenate(
        [coord04.reshape(LVLS - 1, NBATCH, NQ, 4), coord5[None]], axis=0)

    logits5 = cls5.reshape(NBATCH, NFLAT)
    pad = jnp.full((NBATCH, NPAD - NFLAT), -jnp.inf, jnp.float32)
    bits = jnp.concatenate([logits5, pad], axis=1)
    boxes5 = coord5.reshape(NBATCH, NQ * 4)
    img_h = orig_target_sizes[:, 0].astype(jnp.float32)
    img_w = orig_target_sizes[:, 1].astype(jnp.float32)
    wb = jnp.broadcast_to(img_w[:, None], (NBATCH, 16)) * jnp.ones((NBATCH, 16))
    hb = jnp.broadcast_to(img_h[:, None], (NBATCH, 16)) * jnp.ones((NBATCH, 16))
    zz = jnp.zeros((16 * HSTRIDE,), jnp.int32)
    scores_p, labels_p, boxes_p = _run_topk(bits, boxes5, wb, hb, zz)
    boxes_p = boxes_p.reshape(NBATCH, OUTW, 4)
    return (outputs_class, outputs_coord,
            scores_p[:, :KTOP], labels_p[:, :KTOP], boxes_p[:, :KTOP, :])
